# baseline (speedup 1.0000x reference)
import functools

import jax
import jax.numpy as jnp
from jax import lax
from jax.experimental import pallas as pl
from jax.experimental.pallas import tpu as pltpu
from jax.experimental.pallas import tpu_sc as plsc

BATCH = 16384
DIM = 32
CHUNK = 128


def kernel(user_idx, item_idx, user_emb, item_emb):
    info = plsc.get_sparse_core_info()
    nw = info.num_cores * info.num_subcores
    b_per_w = BATCH // nw
    n_chunks = b_per_w // CHUNK

    uidx2 = user_idx.astype(jnp.int32).reshape(nw * n_chunks, CHUNK)
    iidx2 = item_idx.astype(jnp.int32).reshape(nw * n_chunks, CHUNK)

    mesh = plsc.VectorSubcoreMesh(core_axis_name="c", subcore_axis_name="s")

    @functools.partial(
        pl.kernel,
        mesh=mesh,
        compiler_params=pltpu.CompilerParams(use_tc_tiling_on_sc=False),
        out_type=(
            jax.ShapeDtypeStruct((BATCH, DIM), jnp.float32),
            jax.ShapeDtypeStruct((BATCH, DIM), jnp.float32),
        ),
        scratch_types=[
            pltpu.VMEM((n_chunks, CHUNK), jnp.int32),
            pltpu.VMEM((n_chunks, CHUNK), jnp.int32),
            pltpu.VMEM((b_per_w, DIM), jnp.float32),
            pltpu.VMEM((b_per_w, DIM), jnp.float32),
            pltpu.SemaphoreType.DMA,
            pltpu.SemaphoreType.DMA,
        ],
    )
    def mf_gather(uidx_hbm, iidx_hbm, uemb_hbm, iemb_hbm, out_u, out_i,
                  uidx_v, iidx_v, urows, irows, gsem, osem):
        wid = lax.axis_index("s") * info.num_cores + lax.axis_index("c")
        row0 = wid * n_chunks
        pltpu.sync_copy(uidx_hbm.at[pl.ds(row0, n_chunks)], uidx_v)
        pltpu.sync_copy(iidx_hbm.at[pl.ds(row0, n_chunks)], iidx_v)
        copies = []
        for j in range(n_chunks):
            copies.append(pltpu.make_async_copy(
                uemb_hbm.at[uidx_v.at[j]],
                urows.at[pl.ds(j * CHUNK, CHUNK)], gsem))
            copies.append(pltpu.make_async_copy(
                iemb_hbm.at[iidx_v.at[j]],
                irows.at[pl.ds(j * CHUNK, CHUNK)], gsem))
        for c in copies:
            c.start()
        for c in copies:
            c.wait()
        base = wid * b_per_w
        ou = pltpu.make_async_copy(urows, out_u.at[pl.ds(base, b_per_w)], osem)
        oi = pltpu.make_async_copy(irows, out_i.at[pl.ds(base, b_per_w)], osem)
        ou.start()
        oi.start()
        ou.wait()
        oi.wait()

    return mf_gather(uidx2, iidx2, user_emb, item_emb)


# speedup vs baseline: 1.5016x; 1.5016x over previous
import functools

import jax
import jax.numpy as jnp
from jax import lax
from jax.experimental import pallas as pl
from jax.experimental.pallas import tpu as pltpu
from jax.experimental.pallas import tpu_sc as plsc

BATCH = 16384
DIM = 32


def kernel(user_idx, item_idx, user_emb, item_emb):
    info = plsc.get_sparse_core_info()
    nw = info.num_cores * info.num_subcores
    b_per_w = BATCH // nw

    uidx = user_idx.astype(jnp.int32)
    iidx = item_idx.astype(jnp.int32)

    mesh = plsc.VectorSubcoreMesh(core_axis_name="c", subcore_axis_name="s")

    @functools.partial(
        pl.kernel,
        mesh=mesh,
        out_type=(
            jax.ShapeDtypeStruct((BATCH, DIM), jnp.float32),
            jax.ShapeDtypeStruct((BATCH, DIM), jnp.float32),
        ),
        scratch_types=[
            pltpu.VMEM((b_per_w,), jnp.int32),
            pltpu.VMEM((b_per_w,), jnp.int32),
            pltpu.VMEM((b_per_w // 2, DIM), jnp.float32),
            pltpu.VMEM((b_per_w // 2, DIM), jnp.float32),
            pltpu.SemaphoreType.DMA,
            pltpu.SemaphoreType.DMA,
        ],
    )
    def mf_gather(uidx_hbm, iidx_hbm, uemb_hbm, iemb_hbm, out_u, out_i,
                  uidx_v, iidx_v, urows, irows, gsem, osem):
        wid = lax.axis_index("s") * info.num_cores + lax.axis_index("c")
        base = wid * b_per_w
        pltpu.sync_copy(uidx_hbm.at[pl.ds(base, b_per_w)], uidx_v)
        pltpu.sync_copy(iidx_hbm.at[pl.ds(base, b_per_w)], iidx_v)

        half = b_per_w // 2
        for c in range(2):
            def issue(j, _):
                uvec = uidx_v[pl.ds(c * half + j * 16, 16)]
                ivec = iidx_v[pl.ds(c * half + j * 16, 16)]
                for l in range(16):
                    pltpu.make_async_copy(
                        uemb_hbm.at[uvec[l]],
                        urows.at[j * 16 + l], gsem).start()
                    pltpu.make_async_copy(
                        iemb_hbm.at[ivec[l]],
                        irows.at[j * 16 + l], gsem).start()
                return 0

            lax.fori_loop(0, half // 16, issue, 0)
            pltpu.make_async_copy(
                uemb_hbm.at[pl.ds(0, half)], urows, gsem).wait()
            pltpu.make_async_copy(
                iemb_hbm.at[pl.ds(0, half)], irows, gsem).wait()

            ou = pltpu.make_async_copy(
                urows, out_u.at[pl.ds(base + c * half, half)], osem)
            oi = pltpu.make_async_copy(
                irows, out_i.at[pl.ds(base + c * half, half)], osem)
            ou.start()
            oi.start()
            ou.wait()
            oi.wait()

    return mf_gather(uidx, iidx, user_emb, item_emb)
